# baseline (speedup 1.0000x reference)
import functools

import jax
import jax.numpy as jnp
from jax import lax
from jax.experimental import pallas as pl
from jax.experimental.pallas import tpu as pltpu
from jax.experimental.pallas import tpu_sc as plsc

NUM_CORES = 2
NUM_SUBCORES = 16
NUM_WORKERS = NUM_CORES * NUM_SUBCORES
CHUNK = 128


def _emb_call(n_chunks, d):
    mesh = plsc.VectorSubcoreMesh(core_axis_name="c", subcore_axis_name="s")
    rows_per_worker = n_chunks * CHUNK

    @functools.partial(
        pl.kernel,
        out_type=jax.ShapeDtypeStruct((NUM_WORKERS * rows_per_worker, d),
                                      jnp.float32),
        mesh=mesh,
        scratch_types=[
            pltpu.VMEM((n_chunks, CHUNK), jnp.int32),
            pltpu.VMEM((CHUNK, d), jnp.float32),
            pltpu.SemaphoreType.DMA,
        ],
        compiler_params=pltpu.CompilerParams(use_tc_tiling_on_sc=False),
    )
    def emb(idx_hbm, w_hbm, out_hbm, idx_v, rows_v, sem):
        wid = lax.axis_index("s") * NUM_CORES + lax.axis_index("c")
        base = wid * rows_per_worker
        pltpu.sync_copy(idx_hbm.at[wid], idx_v)

        def body(g, carry):
            pltpu.async_copy(w_hbm.at[idx_v.at[g]], rows_v, sem).wait()
            pltpu.sync_copy(rows_v,
                            out_hbm.at[pl.ds(base + g * CHUNK, CHUNK)])
            return carry

        lax.fori_loop(0, n_chunks, body, 0)

    return emb


def kernel(x, weight):
    b0, b1 = x.shape
    n_rows = b0 * b1
    d = weight.shape[1]
    assert n_rows % (NUM_WORKERS * CHUNK) == 0
    n_chunks = n_rows // (NUM_WORKERS * CHUNK)
    idx = x.reshape(NUM_WORKERS, n_chunks, CHUNK).astype(jnp.int32)
    out = _emb_call(n_chunks, d)(idx, weight)
    return out.reshape(b0, b1, d)


# speedup vs baseline: 1.0640x; 1.0640x over previous
import functools

import jax
import jax.numpy as jnp
from jax import lax
from jax.experimental import pallas as pl
from jax.experimental.pallas import tpu as pltpu
from jax.experimental.pallas import tpu_sc as plsc

NUM_CORES = 2
NUM_SUBCORES = 16
NUM_WORKERS = NUM_CORES * NUM_SUBCORES
CHUNK = 128
K = 5
STEP = K * CHUNK


def _emb_call(n_steps, d):
    mesh = plsc.VectorSubcoreMesh(core_axis_name="c", subcore_axis_name="s")
    rows_per_worker = n_steps * STEP
    n_chunks = n_steps * K

    @functools.partial(
        pl.kernel,
        out_type=jax.ShapeDtypeStruct((NUM_WORKERS * rows_per_worker, d),
                                      jnp.float32),
        mesh=mesh,
        scratch_types=[
            pltpu.VMEM((n_chunks, CHUNK), jnp.int32),
            pltpu.VMEM((2, STEP, d), jnp.float32),
            pltpu.SemaphoreType.DMA,
            pltpu.SemaphoreType.DMA,
            pltpu.SemaphoreType.DMA,
            pltpu.SemaphoreType.DMA,
        ],
        compiler_params=pltpu.CompilerParams(use_tc_tiling_on_sc=False),
    )
    def emb(idx_hbm, w_hbm, out_hbm, idx_v, rows_v, g0, g1, w0, w1):
        wid = lax.axis_index("s") * NUM_CORES + lax.axis_index("c")
        base = wid * rows_per_worker
        pltpu.sync_copy(idx_hbm.at[wid], idx_v)
        gsems = (g0, g1)
        wsems = (w0, w1)

        def fire(s, buf):
            for j in range(K):
                pltpu.async_copy(
                    w_hbm.at[idx_v.at[s * K + j]],
                    rows_v.at[buf, pl.ds(j * CHUNK, CHUNK)],
                    gsems[buf])

        def drain_gathers(buf):
            pltpu.make_async_copy(
                out_hbm.at[pl.ds(0, STEP)], rows_v.at[buf],
                gsems[buf]).wait()

        def wait_writeback(s, buf):
            pltpu.make_async_copy(
                rows_v.at[buf],
                out_hbm.at[pl.ds(base + s * STEP, STEP)],
                wsems[buf]).wait()

        def do_step(s, buf):
            drain_gathers(buf)
            pltpu.async_copy(
                rows_v.at[buf],
                out_hbm.at[pl.ds(base + s * STEP, STEP)],
                wsems[buf])
            nxt = buf ^ 1

            @pl.when(s > 0)
            def _():
                wait_writeback(s - 1, nxt)

            @pl.when(s + 1 < n_steps)
            def _():
                fire(s + 1, nxt)

        fire(0, 0)

        def body(i, carry):
            do_step(2 * i, 0)
            do_step(2 * i + 1, 1)
            return carry

        lax.fori_loop(0, n_steps // 2, body, 0)
        wait_writeback(n_steps - 1, (n_steps - 1) % 2)

    return emb


def kernel(x, weight):
    b0, b1 = x.shape
    n_rows = b0 * b1
    d = weight.shape[1]
    assert n_rows % (NUM_WORKERS * STEP) == 0
    n_steps = n_rows // (NUM_WORKERS * STEP)
    assert n_steps % 2 == 0
    idx = x.reshape(NUM_WORKERS, n_steps * K, CHUNK).astype(jnp.int32)
    out = _emb_call(n_steps, d)(idx, weight)
    return out.reshape(b0, b1, d)
